# baseline (speedup 1.0000x reference)
import functools
import jax
import jax.numpy as jnp
from jax import lax
from jax.experimental import pallas as pl
from jax.experimental.pallas import tpu as pltpu
from jax.experimental.pallas import tpu_sc as plsc

BATCH = 4
SEQ = 4096
DIM = 1024
NC = 2
NS = 16
NW = NC * NS
ROWS = BATCH * SEQ
RPW = ROWS // NW
WPB = NW // BATCH
CH = 16
LANES = 16


def _sc_add_kernel(x_hbm, pe_hbm, out_hbm, xbuf, pebuf):
    c = lax.axis_index("c")
    s = lax.axis_index("s")
    wid = c * NS + s
    row0 = wid * RPW
    pe_row0 = lax.rem(wid, WPB) * RPW

    def chunk_body(ci, carry):
        off = (row0 + ci * CH) * DIM
        pe_off = (pe_row0 + ci * CH) * DIM
        pltpu.sync_copy(x_hbm.at[pl.ds(off, CH * DIM)], xbuf)
        pltpu.sync_copy(pe_hbm.at[pl.ds(pe_off, CH * DIM)], pebuf)

        def add_body(i, carry2):
            idx = i * LANES
            xbuf[pl.ds(idx, LANES)] = (
                xbuf[pl.ds(idx, LANES)] + pebuf[pl.ds(idx, LANES)]
            )
            return carry2

        lax.fori_loop(0, CH * DIM // LANES, add_body, 0)
        pltpu.sync_copy(xbuf, out_hbm.at[pl.ds(off, CH * DIM)])
        return carry

    lax.fori_loop(0, RPW // CH, chunk_body, 0)


def kernel(x, pos_table):
    B, S, D = x.shape
    mesh = plsc.VectorSubcoreMesh(core_axis_name="c", subcore_axis_name="s")
    run = functools.partial(
        pl.kernel,
        mesh=mesh,
        out_type=jax.ShapeDtypeStruct((B * S * D,), jnp.float32),
        scratch_types=[
            pltpu.VMEM((CH * DIM,), jnp.float32),
            pltpu.VMEM((CH * DIM,), jnp.float32),
        ],
    )(_sc_add_kernel)
    out = run(x.reshape(-1), pos_table.reshape(-1))
    return out.reshape(B, S, D)


# speedup vs baseline: 1.4553x; 1.4553x over previous
import functools
import jax
import jax.numpy as jnp
from jax import lax
from jax.experimental import pallas as pl
from jax.experimental.pallas import tpu as pltpu
from jax.experimental.pallas import tpu_sc as plsc

BATCH = 4
SEQ = 4096
DIM = 1024
NC = 2
NS = 16
NW = NC * NS
ROWS = BATCH * SEQ
RPW = ROWS // NW
WPB = NW // BATCH
CH = 32
LANES = 16
UNROLL = 8


def _sc_add_kernel(x_hbm, pe_hbm, out_hbm, xbuf, pebuf):
    c = lax.axis_index("c")
    s = lax.axis_index("s")
    wid = c * NS + s
    row0 = wid * RPW
    pe_row0 = lax.rem(wid, WPB) * RPW

    def chunk_body(ci, carry):
        off = (row0 + ci * CH) * DIM
        pe_off = (pe_row0 + ci * CH) * DIM
        pltpu.sync_copy(x_hbm.at[pl.ds(off, CH * DIM)], xbuf)
        pltpu.sync_copy(pe_hbm.at[pl.ds(pe_off, CH * DIM)], pebuf)

        def add_body(i, carry2):
            base = i * (LANES * UNROLL)
            for u in range(UNROLL):
                idx = base + u * LANES
                xbuf[pl.ds(idx, LANES)] = (
                    xbuf[pl.ds(idx, LANES)] + pebuf[pl.ds(idx, LANES)]
                )
            return carry2

        lax.fori_loop(0, CH * DIM // (LANES * UNROLL), add_body, 0)
        pltpu.sync_copy(xbuf, out_hbm.at[pl.ds(off, CH * DIM)])
        return carry

    lax.fori_loop(0, RPW // CH, chunk_body, 0)


def kernel(x, pos_table):
    B, S, D = x.shape
    mesh = plsc.VectorSubcoreMesh(core_axis_name="c", subcore_axis_name="s")
    run = functools.partial(
        pl.kernel,
        mesh=mesh,
        out_type=jax.ShapeDtypeStruct((B * S * D,), jnp.float32),
        scratch_types=[
            pltpu.VMEM((CH * DIM,), jnp.float32),
            pltpu.VMEM((CH * DIM,), jnp.float32),
        ],
    )(_sc_add_kernel)
    out = run(x.reshape(-1), pos_table.reshape(-1))
    return out.reshape(B, S, D)


# speedup vs baseline: 1.8178x; 1.2491x over previous
import functools
import jax
import jax.numpy as jnp
from jax import lax
from jax.experimental import pallas as pl
from jax.experimental.pallas import tpu as pltpu
from jax.experimental.pallas import tpu_sc as plsc

BATCH = 4
SEQ = 4096
DIM = 1024
NC = 2
NS = 16
NW = NC * NS
SPW = SEQ // NW
CH = 16
NCH = SPW // CH
NT = NCH * BATCH
CHW = CH * DIM
LANES = 16
UNROLL = 8


def _sc_add_kernel(x_hbm, pe_hbm, out_hbm,
                   xb0, xb1, pb0, pb1, sx0, sx1, sp0, sp1, so0, so1):
    c = lax.axis_index("c")
    s = lax.axis_index("s")
    wid = c * NS + s
    pe_base = wid * (SPW * DIM)
    xbufs = (xb0, xb1)
    pebufs = (pb0, pb1)
    sxs = (sx0, sx1)
    sps = (sp0, sp1)
    sos = (so0, so1)

    def x_off(t):
        ci, b = divmod(t, BATCH)
        return (b * SEQ * DIM) + pe_base + ci * CHW

    def pe_off(ci):
        return pe_base + ci * CHW

    pltpu.async_copy(x_hbm.at[pl.ds(x_off(0), CHW)], xbufs[0], sxs[0])
    pltpu.async_copy(pe_hbm.at[pl.ds(pe_off(0), CHW)], pebufs[0], sps[0])

    for t in range(NT):
        xq = t % 2
        ci, b = divmod(t, BATCH)
        pq = ci % 2
        if t + 1 < NT:
            if t >= 1:
                pltpu.make_async_copy(
                    xbufs[1 - xq], out_hbm.at[pl.ds(x_off(t - 1), CHW)],
                    sos[1 - xq]).wait()
            pltpu.async_copy(
                x_hbm.at[pl.ds(x_off(t + 1), CHW)], xbufs[1 - xq],
                sxs[1 - xq])
        if b == 0 and ci + 1 < NCH:
            pltpu.async_copy(
                pe_hbm.at[pl.ds(pe_off(ci + 1), CHW)], pebufs[1 - pq],
                sps[1 - pq])
        pltpu.make_async_copy(
            x_hbm.at[pl.ds(x_off(t), CHW)], xbufs[xq], sxs[xq]).wait()
        if b == 0:
            pltpu.make_async_copy(
                pe_hbm.at[pl.ds(pe_off(ci), CHW)], pebufs[pq], sps[pq]).wait()

        xb = xbufs[xq]
        pb = pebufs[pq]

        @plsc.parallel_loop(0, CHW // LANES, 1, unroll=UNROLL)
        def _(i):
            idx = i * LANES
            xb[pl.ds(idx, LANES)] = xb[pl.ds(idx, LANES)] + pb[pl.ds(idx, LANES)]

        pltpu.async_copy(xbufs[xq], out_hbm.at[pl.ds(x_off(t), CHW)], sos[xq])

    pltpu.make_async_copy(
        xbufs[0], out_hbm.at[pl.ds(x_off(NT - 2), CHW)], sos[0]).wait()
    pltpu.make_async_copy(
        xbufs[1], out_hbm.at[pl.ds(x_off(NT - 1), CHW)], sos[1]).wait()


def kernel(x, pos_table):
    B, S, D = x.shape
    mesh = plsc.VectorSubcoreMesh(core_axis_name="c", subcore_axis_name="s")
    run = functools.partial(
        pl.kernel,
        mesh=mesh,
        out_type=jax.ShapeDtypeStruct((B * S * D,), jnp.float32),
        scratch_types=[
            pltpu.VMEM((CHW,), jnp.float32),
            pltpu.VMEM((CHW,), jnp.float32),
            pltpu.VMEM((CHW,), jnp.float32),
            pltpu.VMEM((CHW,), jnp.float32),
            pltpu.SemaphoreType.DMA,
            pltpu.SemaphoreType.DMA,
            pltpu.SemaphoreType.DMA,
            pltpu.SemaphoreType.DMA,
            pltpu.SemaphoreType.DMA,
            pltpu.SemaphoreType.DMA,
        ],
    )(_sc_add_kernel)
    out = run(x.reshape(-1), pos_table.reshape(-1))
    return out.reshape(B, S, D)


# speedup vs baseline: 4.7428x; 2.6091x over previous
import functools
import jax
import jax.numpy as jnp
from jax import lax
from jax.experimental import pallas as pl
from jax.experimental.pallas import tpu as pltpu
from jax.experimental.pallas import tpu_sc as plsc

BATCH = 4
SEQ = 4096
DIM = 1024
NC = 2
NS = 16
NW = NC * NS
SPW = SEQ // NW
CH = 16
NCH = SPW // CH
NT = NCH * BATCH
LANES = 16
UNROLL = 8


def _sc_add_kernel(x_hbm, pe_hbm, out_hbm,
                   xb0, xb1, pb0, pb1, sx0, sx1, sp0, sp1, so0, so1):
    c = lax.axis_index("c")
    s = lax.axis_index("s")
    wid = c * NS + s
    s_base = wid * SPW
    xbufs = (xb0, xb1)
    pebufs = (pb0, pb1)
    sxs = (sx0, sx1)
    sps = (sp0, sp1)
    sos = (so0, so1)

    def x_slice(t):
        ci, b = divmod(t, BATCH)
        return (b, pl.ds(s_base + ci * CH, CH), slice(None))

    def pe_slice(ci):
        return (pl.ds(s_base + ci * CH, CH), slice(None))

    pltpu.async_copy(x_hbm.at[x_slice(0)], xbufs[0], sxs[0])
    pltpu.async_copy(pe_hbm.at[pe_slice(0)], pebufs[0], sps[0])

    for t in range(NT):
        xq = t % 2
        ci, b = divmod(t, BATCH)
        pq = ci % 2
        if t + 1 < NT:
            if t >= 1:
                pltpu.make_async_copy(
                    xbufs[1 - xq], out_hbm.at[x_slice(t - 1)],
                    sos[1 - xq]).wait()
            pltpu.async_copy(x_hbm.at[x_slice(t + 1)], xbufs[1 - xq],
                             sxs[1 - xq])
        if b == 0 and ci + 1 < NCH:
            pltpu.async_copy(pe_hbm.at[pe_slice(ci + 1)], pebufs[1 - pq],
                             sps[1 - pq])
        pltpu.make_async_copy(x_hbm.at[x_slice(t)], xbufs[xq], sxs[xq]).wait()
        if b == 0:
            pltpu.make_async_copy(pe_hbm.at[pe_slice(ci)], pebufs[pq],
                                  sps[pq]).wait()

        xb = xbufs[xq]
        pb = pebufs[pq]

        def row_body(r, carry):
            @plsc.parallel_loop(0, DIM // LANES, 1, unroll=UNROLL)
            def _(i):
                idx = i * LANES
                xb[r, pl.ds(idx, LANES)] = (
                    xb[r, pl.ds(idx, LANES)] + pb[r, pl.ds(idx, LANES)]
                )
            return carry

        lax.fori_loop(0, CH, row_body, 0)

        pltpu.async_copy(xbufs[xq], out_hbm.at[x_slice(t)], sos[xq])

    pltpu.make_async_copy(
        xbufs[0], out_hbm.at[x_slice(NT - 2)], sos[0]).wait()
    pltpu.make_async_copy(
        xbufs[1], out_hbm.at[x_slice(NT - 1)], sos[1]).wait()


def kernel(x, pos_table):
    B, S, D = x.shape
    mesh = plsc.VectorSubcoreMesh(core_axis_name="c", subcore_axis_name="s")
    run = functools.partial(
        pl.kernel,
        mesh=mesh,
        out_type=jax.ShapeDtypeStruct((B, S, D), jnp.float32),
        scratch_types=[
            pltpu.VMEM((CH, DIM), jnp.float32),
            pltpu.VMEM((CH, DIM), jnp.float32),
            pltpu.VMEM((CH, DIM), jnp.float32),
            pltpu.VMEM((CH, DIM), jnp.float32),
            pltpu.SemaphoreType.DMA,
            pltpu.SemaphoreType.DMA,
            pltpu.SemaphoreType.DMA,
            pltpu.SemaphoreType.DMA,
            pltpu.SemaphoreType.DMA,
            pltpu.SemaphoreType.DMA,
        ],
        compiler_params=pltpu.CompilerParams(use_tc_tiling_on_sc=True),
    )(_sc_add_kernel)
    return run(x, pos_table)
